# baseline (speedup 1.0000x reference)
import functools

import jax
import jax.numpy as jnp
from jax import lax
from jax.experimental import pallas as pl
from jax.experimental.pallas import tpu as pltpu
from jax.experimental.pallas import tpu_sc as plsc

B = 16384
D = 64
L = 16
NC = 2
NS = 16
NW = NC * NS
BPW = B // NW
CHUNK = 128
NCHUNK = BPW // CHUNK

_mesh = plsc.VectorSubcoreMesh(core_axis_name="c", subcore_axis_name="s")


@functools.partial(
    pl.kernel,
    mesh=_mesh,
    compiler_params=pltpu.CompilerParams(
        needs_layout_passes=False, use_tc_tiling_on_sc=False),
    out_type=jax.ShapeDtypeStruct((B,), jnp.float32),
    scratch_types=[
        pltpu.VMEM((NCHUNK, CHUNK), jnp.int32),
        pltpu.VMEM((NCHUNK, CHUNK), jnp.int32),
        pltpu.VMEM((BPW, D), jnp.float32),
        pltpu.VMEM((BPW, D), jnp.float32),
        pltpu.VMEM((D,), jnp.float32),
        pltpu.VMEM((L,), jnp.float32),
        pltpu.VMEM((BPW + L,), jnp.float32),
        pltpu.SemaphoreType.DMA,
    ],
)
def _gmf_sc(uid_hbm, iid_hbm, utab_hbm, itab_hbm, w_hbm, bias_hbm, out_hbm,
            uidx, iidx, urows, irows, wv, bv, outv, sem):
    wid = lax.axis_index("s") * NC + lax.axis_index("c")

    pltpu.sync_copy(uid_hbm.at[wid], uidx)
    pltpu.sync_copy(iid_hbm.at[wid], iidx)
    pltpu.sync_copy(w_hbm, wv)
    pltpu.sync_copy(bias_hbm, bv)

    copies = []
    for c in range(NCHUNK):
        copies.append(pltpu.async_copy(
            utab_hbm.at[uidx.at[c]], urows.at[pl.ds(c * CHUNK, CHUNK)], sem))
        copies.append(pltpu.async_copy(
            itab_hbm.at[iidx.at[c]], irows.at[pl.ds(c * CHUNK, CHUNK)], sem))
    for cp in copies:
        cp.wait()

    w_slices = [wv[pl.ds(L * j, L)] for j in range(D // L)]
    bias = bv[...]
    lane0 = lax.iota(jnp.int32, L) == 0

    def rowsum(r, carry):
        acc = bias
        for j in range(D // L):
            u = urows[r, pl.ds(L * j, L)]
            it = irows[r, pl.ds(L * j, L)]
            acc = acc + u * it * w_slices[j]
        sv = jnp.full((L,), jnp.sum(acc), jnp.float32)
        plsc.store_compressed(outv.at[pl.ds(r, L)], sv, mask=lane0)
        return carry

    lax.fori_loop(0, BPW, rowsum, 0)

    pltpu.sync_copy(outv.at[pl.ds(0, BPW)], out_hbm.at[pl.ds(wid * BPW, BPW)])


def kernel(user_ids, item_ids, user_table, item_table, W, b):
    uid = user_ids.astype(jnp.int32).reshape(NW, NCHUNK, CHUNK)
    iid = item_ids.astype(jnp.int32).reshape(NW, NCHUNK, CHUNK)
    w64 = W.reshape(D).astype(jnp.float32)
    bias = jnp.zeros((L,), dtype=jnp.float32).at[0].set(b[0])
    return _gmf_sc(uid, iid, user_table, item_table, w64, bias)
